# baseline (speedup 1.0000x reference)
import functools

import jax
import jax.numpy as jnp
from jax import lax
from jax.experimental import pallas as pl
from jax.experimental.pallas import tpu as pltpu
from jax.experimental.pallas import tpu_sc as plsc

NC, NS = 2, 16
NW = NC * NS
CHUNK = 128
LANES = 16


def _sc_body(rpw, nch, d, sub, table, idx, pos, out, idx_v, rows_v, pos_v, gsem, psem):
    wid = lax.axis_index("s") * NC + lax.axis_index("c")
    base = wid * rpw
    seq = pos.shape[0]
    p0 = (wid % (seq // rpw)) * rpw
    pltpu.sync_copy(idx.at[wid], idx_v)
    nsub = rpw // sub
    csub = sub // CHUNK
    for sb in range(nsub):
        pcopy = pltpu.async_copy(
            pos.at[pl.ds(p0 + sb * sub, sub)], pos_v, psem
        )
        gathers = [
            pltpu.async_copy(
                table.at[idx_v.at[sb * csub + j]],
                rows_v.at[pl.ds(j * CHUNK, CHUNK)],
                gsem,
            )
            for j in range(csub)
        ]
        for g in gathers:
            g.wait()
        pcopy.wait()

        def add_row(i, carry):
            for j in range(d // LANES):
                sl = pl.ds(j * LANES, LANES)
                plsc.addupdate(rows_v.at[i, sl], pos_v[i, sl])
            return carry

        lax.fori_loop(0, sub, add_row, 0)
        pltpu.sync_copy(rows_v, out.at[pl.ds(base + sb * sub, sub)])


@jax.jit
def _embed(idx3, table, pos):
    nw, nch, chunk = idx3.shape
    rpw = nch * chunk
    d = table.shape[1]
    mesh = plsc.VectorSubcoreMesh(
        core_axis_name="c", subcore_axis_name="s", num_cores=NC, num_subcores=NS
    )
    sub = 512
    f = pl.kernel(
        functools.partial(_sc_body, rpw, nch, d, sub),
        out_type=jax.ShapeDtypeStruct((nw * rpw, d), jnp.float32),
        mesh=mesh,
        scratch_types=[
            pltpu.VMEM((nch, chunk), jnp.int32),
            pltpu.VMEM((sub, d), jnp.float32),
            pltpu.VMEM((sub, d), jnp.float32),
            pltpu.SemaphoreType.DMA,
            pltpu.SemaphoreType.DMA,
        ],
        compiler_params=pltpu.CompilerParams(use_tc_tiling_on_sc=False),
    )
    return f(table, idx3, pos)


def kernel(token_indices, token_table, pos_table):
    b, s = token_indices.shape
    rows = b * s
    rpw = rows // NW
    assert rows % NW == 0 and rpw % CHUNK == 0 and s % rpw == 0
    idx3 = token_indices.astype(jnp.int32).reshape(NW, rpw // CHUNK, CHUNK)
    out = _embed(idx3, token_table, pos_table)
    return out.reshape(b, s, token_table.shape[1])


# speedup vs baseline: 1.1074x; 1.1074x over previous
import functools

import jax
import jax.numpy as jnp
from jax import lax
from jax.experimental import pallas as pl
from jax.experimental.pallas import tpu as pltpu
from jax.experimental.pallas import tpu_sc as plsc

NC, NS = 2, 16
NW = NC * NS
LANES = 16
SUB = 128


def _sc_body(rpw, d, t2, idx, pos, out, idx_v, grows_v, pos_v, out_v,
             gsem, psem):
    wid = lax.axis_index("s") * NC + lax.axis_index("c")
    base = wid * rpw
    seq = pos.shape[0]
    p0 = (wid % (seq // rpw)) * rpw
    nsub = rpw // SUB
    pltpu.sync_copy(idx.at[wid], idx_v)
    for sb in range(nsub):
        pcopy = pltpu.async_copy(pos.at[pl.ds(p0 + sb * SUB, SUB)], pos_v, psem)
        pltpu.async_copy(t2.at[idx_v.at[sb]], grows_v, gsem).wait()
        pcopy.wait()

        def add_row(i, carry):
            for q in range(d // LANES):
                sl = pl.ds(q * LANES, LANES)
                out_v[i, sl] = grows_v[i, sl] + pos_v[i, sl]
            return carry

        lax.fori_loop(0, SUB, add_row, 0)
        pltpu.sync_copy(out_v, out.at[pl.ds(base + sb * SUB, SUB)])


@jax.jit
def _embed(idx3, t2, pos):
    nw, nsub, sub = idx3.shape
    rpw = nsub * sub
    d = t2.shape[1] // 2
    mesh = plsc.VectorSubcoreMesh(
        core_axis_name="c", subcore_axis_name="s", num_cores=NC, num_subcores=NS
    )
    f = pl.kernel(
        functools.partial(_sc_body, rpw, d),
        out_type=jax.ShapeDtypeStruct((nw * rpw, d), jnp.float32),
        mesh=mesh,
        scratch_types=[
            pltpu.VMEM((nsub, sub), jnp.int32),
            pltpu.VMEM((sub, 2 * d), jnp.float32),
            pltpu.VMEM((sub, d), jnp.float32),
            pltpu.VMEM((sub, d), jnp.float32),
            pltpu.SemaphoreType.DMA,
            pltpu.SemaphoreType.DMA,
        ],
        compiler_params=pltpu.CompilerParams(use_tc_tiling_on_sc=True),
    )
    return f(t2, idx3, pos)


def kernel(token_indices, token_table, pos_table):
    b, s = token_indices.shape
    rows = b * s
    rpw = rows // NW
    v, d = token_table.shape
    assert rows % NW == 0 and rpw % SUB == 0 and s % rpw == 0 and d == 64
    t2 = jnp.pad(token_table, ((0, 0), (0, d)))
    idx3 = token_indices.astype(jnp.int32).reshape(NW, rpw // SUB, SUB)
    out = _embed(idx3, t2, pos_table)
    return out.reshape(b, s, d)
